# Initial kernel scaffold; baseline (speedup 1.0000x reference)
import jax
import jax.numpy as jnp
from jax.experimental import pallas as pl


def kernel(msa, pair, state, xyz, chirals, dist_matrix, is_atom, atom_frames, idx, bond_feats, W_node, b_node, W_edge, b_edge, W_msg, b_msg, W_xyz, b_xyz, W_state, b_state, W_sc1, b_sc1, W_sc2, b_sc2):
    raise NotImplementedError("write your pallas kernel here")


# speedup vs baseline: 3.0129x; 3.0129x over previous
import functools

import jax
import jax.numpy as jnp
from jax import lax
from jax.experimental import pallas as pl
from jax.experimental.pallas import tpu as pltpu
from jax.experimental.pallas import tpu_sc as plsc

L = 1024
K = 32
D = 64
D_HID = 128
N_TORSION = 10
BLK = 128
NBLK = L // BLK
NW = 32
RPW = L * K // NW
CHUNK = 128
NCH = RPW // CHUNK
NPH = 2
CPP = NCH // NPH
_PREC = lax.Precision.DEFAULT
_HI = lax.Precision.HIGHEST


def _make_sc_gather():
    mesh = plsc.VectorSubcoreMesh(core_axis_name="c", subcore_axis_name="s")
    rpp = RPW // NPH

    @functools.partial(
        pl.kernel,
        out_type=jax.ShapeDtypeStruct((L * K, 128), jnp.float32),
        mesh=mesh,
        scratch_types=[
            pltpu.VMEM((NCH, CHUNK), jnp.int32),
            pltpu.VMEM((rpp, 128), jnp.float32),
            pltpu.SemaphoreType.DMA,
        ],
    )
    def gather(table_hbm, idx_hbm, out_hbm, idx_v, rows_v, sem):
        w = lax.axis_index("s") * 2 + lax.axis_index("c")
        pltpu.sync_copy(idx_hbm.at[w], idx_v)
        for ph in range(NPH):
            cps = [
                pltpu.async_copy(
                    table_hbm.at[idx_v.at[ph * CPP + c]],
                    rows_v.at[pl.ds(c * CHUNK, CHUNK)],
                    sem,
                )
                for c in range(CPP)
            ]
            for cp in cps:
                cp.wait()
            pltpu.sync_copy(rows_v, out_hbm.at[pl.ds(w * RPW + ph * rpp, rpp)])

    return gather


_SC_CACHE = {}


def _gather_rows128(table, idx3):
    if "g" not in _SC_CACHE:
        _SC_CACHE["g"] = _make_sc_gather()
    return _SC_CACHE["g"](table, idx3)


def _knn_body(ca_ref, cat_ref, dist_ref, nbr_ref, prow_ref, par_ref,
              d2n_ref, ss_ref, dmn_ref, w_scr):
    i0 = pl.program_id(0) * BLK
    ca = ca_ref[...]
    cat = cat_ref[...]
    d2 = (
        jnp.square(ca[:, 0:1] - cat[0:1, :])
        + jnp.square(ca[:, 1:2] - cat[1:2, :])
    ) + jnp.square(ca[:, 2:3] - cat[2:3, :])
    col = lax.broadcasted_iota(jnp.int32, (BLK, L), 1)
    row = i0 + lax.broadcasted_iota(jnp.int32, (BLK, L), 0)
    w_scr[...] = jnp.where(col == row, jnp.float32(jnp.inf), d2)
    dist = dist_ref[...]
    nbrs, d2s, dms = [], [], []
    for _ in range(K):
        w = w_scr[...]
        mv = jnp.min(w, axis=1, keepdims=True)
        am = jnp.min(jnp.where(w <= mv, col, L), axis=1, keepdims=True)
        w_scr[...] = jnp.where(col == am, jnp.float32(jnp.inf), w)
        dms.append(jnp.sum(jnp.where(col == am, dist, 0.0), axis=1,
                           keepdims=True))
        nbrs.append(am)
        d2s.append(mv)
    nbr = jnp.concatenate(nbrs, axis=1)
    d2n_ref[...] = jnp.concatenate(d2s, axis=1)
    dmn_ref[...] = jnp.concatenate(dms, axis=1)
    nbr_ref[...] = nbr
    rowk = i0 + lax.broadcasted_iota(jnp.int32, (BLK, K), 0)
    prow_ref[...] = rowk * (L // 2) + (nbr >> 1)
    par_ref[...] = (nbr & 1).astype(jnp.float32)
    ss_ref[...] = (rowk - nbr).astype(jnp.float32) / 100.0


def _knn(ca_pad, ca_t, dist2d):
    return pl.pallas_call(
        _knn_body,
        grid=(NBLK,),
        in_specs=[
            pl.BlockSpec((BLK, 8), lambda i: (i, 0)),
            pl.BlockSpec((8, L), lambda i: (0, 0)),
            pl.BlockSpec((BLK, L), lambda i: (i, 0)),
        ],
        out_specs=[pl.BlockSpec((BLK, K), lambda i: (i, 0))] * 6,
        out_shape=[
            jax.ShapeDtypeStruct((L, K), jnp.int32),
            jax.ShapeDtypeStruct((L, K), jnp.int32),
            jax.ShapeDtypeStruct((L, K), jnp.float32),
            jax.ShapeDtypeStruct((L, K), jnp.float32),
            jax.ShapeDtypeStruct((L, K), jnp.float32),
            jax.ShapeDtypeStruct((L, K), jnp.float32),
        ],
        scratch_shapes=[pltpu.VMEM((BLK, L), jnp.float32)],
    )(ca_pad, ca_t, dist2d)


def _edge_body(pairg_ref, par_ref, d2n_ref, ss_ref, dm_ref, cen_ref, we_ref,
               be_ref, msa_ref, st_ref, wn_ref, bn_ref, e_ref, h_ref):
    p = par_ref[...]
    pg = pairg_ref[...]
    pair_nbr = pg[:, :D] * (1.0 - p) + pg[:, D:] * p
    d = jnp.sqrt(d2n_ref[...] + 1e-8)
    z = (d - cen_ref[...]) / jnp.float32(20.0 / D)
    rbf = jnp.exp(-z * z)
    e_in = jnp.concatenate([pair_nbr, rbf, ss_ref[...], dm_ref[...]], axis=1)
    e_ref[...] = jnp.dot(e_in, we_ref[...], precision=_PREC) + be_ref[...]
    hin = jnp.concatenate([msa_ref[...], st_ref[...]], axis=1)
    h0 = jnp.dot(hin, wn_ref[...], precision=_PREC) + bn_ref[...]
    h_ref[...] = jnp.concatenate([h0, jnp.zeros((BLK, D), jnp.float32)],
                                 axis=1)


def _edge_node(pairg, par_f, d2n_f, ss_f, dm_f, centers, we, be,
               msa0, state, wn, bn):
    g = BLK * K
    return pl.pallas_call(
        _edge_body,
        grid=(NBLK,),
        in_specs=[
            pl.BlockSpec((g, 128), lambda i: (i, 0)),
            pl.BlockSpec((g, 1), lambda i: (i, 0)),
            pl.BlockSpec((g, 1), lambda i: (i, 0)),
            pl.BlockSpec((g, 1), lambda i: (i, 0)),
            pl.BlockSpec((g, 1), lambda i: (i, 0)),
            pl.BlockSpec((1, D), lambda i: (0, 0)),
            pl.BlockSpec((2 * D + 2, D), lambda i: (0, 0)),
            pl.BlockSpec((1, D), lambda i: (0, 0)),
            pl.BlockSpec((BLK, D), lambda i: (i, 0)),
            pl.BlockSpec((BLK, D), lambda i: (i, 0)),
            pl.BlockSpec((2 * D, D), lambda i: (0, 0)),
            pl.BlockSpec((1, D), lambda i: (0, 0)),
        ],
        out_specs=[
            pl.BlockSpec((g, D), lambda i: (i, 0)),
            pl.BlockSpec((BLK, 128), lambda i: (i, 0)),
        ],
        out_shape=[
            jax.ShapeDtypeStruct((L * K, D), jnp.float32),
            jax.ShapeDtypeStruct((L, 128), jnp.float32),
        ],
    )(pairg, par_f, d2n_f, ss_f, dm_f, centers, we, be, msa0, state, wn, bn)


def _msg_body(h_ref, hn_ref, e_ref, wm_ref, bm_ref, ho_ref):
    g = BLK * K
    h = h_ref[:, :D]
    h_self = jnp.broadcast_to(h[:, None, :], (BLK, K, D)).reshape(g, D)
    m_in = jnp.concatenate([h_self, hn_ref[:, :D], e_ref[...]], axis=1)
    m = jnp.maximum(
        jnp.dot(m_in, wm_ref[...], precision=_PREC) + bm_ref[...], 0.0)
    mean = jnp.mean(m.reshape(BLK, K, D), axis=1)
    hnew = h + mean
    ho_ref[...] = jnp.concatenate([hnew, jnp.zeros((BLK, D), jnp.float32)],
                                  axis=1)


def _msg(h, hn, e, wm, bm):
    g = BLK * K
    return pl.pallas_call(
        _msg_body,
        grid=(NBLK,),
        in_specs=[
            pl.BlockSpec((BLK, 128), lambda i: (i, 0)),
            pl.BlockSpec((g, 128), lambda i: (i, 0)),
            pl.BlockSpec((g, D), lambda i: (i, 0)),
            pl.BlockSpec((3 * D, D), lambda i: (0, 0)),
            pl.BlockSpec((1, D), lambda i: (0, 0)),
        ],
        out_specs=pl.BlockSpec((BLK, 128), lambda i: (i, 0)),
        out_shape=jax.ShapeDtypeStruct((L, 128), jnp.float32),
    )(h, hn, e, wm, bm)


def _tail_body(h_ref, xyz_ref, st_ref, msa_ref, wx_ref, bx_ref, wst_ref,
               bst_ref, w1_ref, b1_ref, w2_ref, b2_ref, msel_ref,
               xo_ref, so_ref, ao_ref):
    h = h_ref[:, :D]
    disp = jnp.dot(h, wx_ref[...], precision=_PREC) + bx_ref[...]
    xo_ref[...] = xyz_ref[...] + 0.1 * disp
    sn = st_ref[...] + jnp.dot(h, wst_ref[...], precision=_PREC) + bst_ref[...]
    so_ref[...] = sn
    ain = jnp.concatenate([msa_ref[...], sn], axis=1)
    hid = jnp.maximum(jnp.dot(ain, w1_ref[...], precision=_PREC) + b1_ref[...], 0.0)
    al = jnp.dot(hid, w2_ref[...], precision=_PREC) + b2_ref[...]
    ps = jnp.dot(al * al, msel_ref[...], precision=_HI)
    ao_ref[...] = al / (jnp.sqrt(ps) + 1e-8)


def _tail(h, xyz9, state, msa0, wx, bx, wst, bst, w1, b1, w2, b2, msel):
    t2 = 2 * N_TORSION
    return pl.pallas_call(
        _tail_body,
        grid=(NBLK,),
        in_specs=[
            pl.BlockSpec((BLK, 128), lambda i: (i, 0)),
            pl.BlockSpec((BLK, 9), lambda i: (i, 0)),
            pl.BlockSpec((BLK, D), lambda i: (i, 0)),
            pl.BlockSpec((BLK, D), lambda i: (i, 0)),
            pl.BlockSpec((D, 9), lambda i: (0, 0)),
            pl.BlockSpec((1, 9), lambda i: (0, 0)),
            pl.BlockSpec((D, D), lambda i: (0, 0)),
            pl.BlockSpec((1, D), lambda i: (0, 0)),
            pl.BlockSpec((2 * D, D_HID), lambda i: (0, 0)),
            pl.BlockSpec((1, D_HID), lambda i: (0, 0)),
            pl.BlockSpec((D_HID, t2), lambda i: (0, 0)),
            pl.BlockSpec((1, t2), lambda i: (0, 0)),
            pl.BlockSpec((t2, t2), lambda i: (0, 0)),
        ],
        out_specs=[
            pl.BlockSpec((BLK, 9), lambda i: (i, 0)),
            pl.BlockSpec((BLK, D), lambda i: (i, 0)),
            pl.BlockSpec((BLK, t2), lambda i: (i, 0)),
        ],
        out_shape=[
            jax.ShapeDtypeStruct((L, 9), jnp.float32),
            jax.ShapeDtypeStruct((L, D), jnp.float32),
            jax.ShapeDtypeStruct((L, t2), jnp.float32),
        ],
    )(h, xyz9, state, msa0, wx, bx, wst, bst, w1, b1, w2, b2, msel)


def kernel(msa, pair, state, xyz, chirals, dist_matrix, is_atom, atom_frames,
           idx, bond_feats, W_node, b_node, W_edge, b_edge, W_msg, b_msg,
           W_xyz, b_xyz, W_state, b_state, W_sc1, b_sc1, W_sc2, b_sc2):
    msa0 = msa[0, 0]
    pair128 = pair.reshape(L * L // 2, 128)
    dist2d = dist_matrix.reshape(L, L)
    st = state[0]
    xyz9 = xyz.reshape(L, 9)

    centers = jnp.linspace(0.0, 20.0, D).reshape(1, D)
    be = b_edge.reshape(1, D)
    bn = b_node.reshape(1, D)
    bx = b_xyz.reshape(1, 9)
    bst = b_state.reshape(1, D)
    b1 = b_sc1.reshape(1, D_HID)
    b2 = b_sc2.reshape(1, 2 * N_TORSION)
    t2 = 2 * N_TORSION
    msel = (jnp.arange(t2)[:, None] // 2 == jnp.arange(t2)[None, :] // 2
            ).astype(jnp.float32)

    xyzs, alphas = [], []
    for _ in range(2):
        ca = xyz9[:, 3:6]
        ca_pad = jnp.pad(ca, ((0, 0), (0, 5)))
        ca_t = jnp.pad(ca.T, ((0, 5), (0, 0)))
        nbr, prow, par, d2n, ss, dmn = _knn(ca_pad, ca_t, dist2d)
        pairg = _gather_rows128(pair128, prow.reshape(NW, NCH, CHUNK))
        e, h = _edge_node(pairg, par.reshape(L * K, 1), d2n.reshape(L * K, 1),
                          ss.reshape(L * K, 1), dmn.reshape(L * K, 1),
                          centers, W_edge, be, msa0, st, W_node, bn)
        nidx3 = nbr.reshape(NW, NCH, CHUNK)
        for l in range(2):
            hn = _gather_rows128(h, nidx3)
            h = _msg(h, hn, e, W_msg[l], b_msg[l].reshape(1, D))
        xyz9, st, al = _tail(h, xyz9, st, msa0, W_xyz, bx, W_state, bst,
                             W_sc1, b1, W_sc2, b2, msel)
        xyzs.append(xyz9.reshape(1, L, 3, 3))
        alphas.append(al.reshape(1, L, N_TORSION, 2))

    return (jnp.stack(xyzs, axis=0), st[None], jnp.stack(alphas, axis=0))


# speedup vs baseline: 3.0838x; 1.0236x over previous
import functools

import jax
import jax.numpy as jnp
from jax import lax
from jax.experimental import pallas as pl
from jax.experimental.pallas import tpu as pltpu
from jax.experimental.pallas import tpu_sc as plsc

L = 1024
K = 32
D = 64
D_HID = 128
N_TORSION = 10
BLK = 128
NBLK = L // BLK
NW = 32
RPW = L * K // NW
CHUNK = 128
NCH = RPW // CHUNK
NPH = 2
CPP = NCH // NPH
_PREC = lax.Precision.DEFAULT
_HI = lax.Precision.HIGHEST


def _make_sc_gather():
    mesh = plsc.VectorSubcoreMesh(core_axis_name="c", subcore_axis_name="s")
    rpp = RPW // NPH

    @functools.partial(
        pl.kernel,
        out_type=jax.ShapeDtypeStruct((L * K, 128), jnp.float32),
        mesh=mesh,
        scratch_types=[
            pltpu.VMEM((NCH, CHUNK), jnp.int32),
            pltpu.VMEM((rpp, 128), jnp.float32),
            pltpu.SemaphoreType.DMA,
        ],
    )
    def gather(table_hbm, idx_hbm, out_hbm, idx_v, rows_v, sem):
        w = lax.axis_index("s") * 2 + lax.axis_index("c")
        pltpu.sync_copy(idx_hbm.at[w], idx_v)
        for ph in range(NPH):
            cps = [
                pltpu.async_copy(
                    table_hbm.at[idx_v.at[ph * CPP + c]],
                    rows_v.at[pl.ds(c * CHUNK, CHUNK)],
                    sem,
                )
                for c in range(CPP)
            ]
            for cp in cps:
                cp.wait()
            pltpu.sync_copy(rows_v, out_hbm.at[pl.ds(w * RPW + ph * rpp, rpp)])

    return gather


def _make_sc_gather_dense():
    mesh = plsc.VectorSubcoreMesh(core_axis_name="c", subcore_axis_name="s")

    @functools.partial(
        pl.kernel,
        out_type=jax.ShapeDtypeStruct((L * K, D), jnp.float32),
        mesh=mesh,
        compiler_params=pltpu.CompilerParams(use_tc_tiling_on_sc=False),
        scratch_types=[
            pltpu.VMEM((NCH, CHUNK), jnp.int32),
            pltpu.VMEM((RPW, D), jnp.float32),
            pltpu.SemaphoreType.DMA,
        ],
    )
    def gather(table_hbm, idx_hbm, out_hbm, idx_v, rows_v, sem):
        w = lax.axis_index("s") * 2 + lax.axis_index("c")
        pltpu.sync_copy(idx_hbm.at[w], idx_v)
        cps = [
            pltpu.async_copy(
                table_hbm.at[idx_v.at[c]],
                rows_v.at[pl.ds(c * CHUNK, CHUNK)],
                sem,
            )
            for c in range(NCH)
        ]
        for cp in cps:
            cp.wait()
        pltpu.sync_copy(rows_v, out_hbm.at[pl.ds(w * RPW, RPW)])

    return gather


_SC_CACHE = {}


def _gather_rows128(table, idx3):
    if "g" not in _SC_CACHE:
        _SC_CACHE["g"] = _make_sc_gather()
    return _SC_CACHE["g"](table, idx3)


def _gather_rows64(table, idx3):
    if "d" not in _SC_CACHE:
        _SC_CACHE["d"] = _make_sc_gather_dense()
    return _SC_CACHE["d"](table, idx3)


def _knn_body(ca_ref, cat_ref, dist_ref, nbr_ref, flat_ref,
              d2n_ref, ss_ref, dmn_ref, w_scr):
    i0 = pl.program_id(0) * BLK
    ca = ca_ref[...]
    cat = cat_ref[...]
    d2 = (
        jnp.square(ca[:, 0:1] - cat[0:1, :])
        + jnp.square(ca[:, 1:2] - cat[1:2, :])
    ) + jnp.square(ca[:, 2:3] - cat[2:3, :])
    col = lax.broadcasted_iota(jnp.int32, (BLK, L), 1)
    row = i0 + lax.broadcasted_iota(jnp.int32, (BLK, L), 0)
    w_scr[...] = jnp.where(col == row, jnp.float32(jnp.inf), d2)
    dist = dist_ref[...]
    nbrs, d2s, dms = [], [], []
    for _ in range(K):
        w = w_scr[...]
        mv = jnp.min(w, axis=1, keepdims=True)
        am = jnp.min(jnp.where(w <= mv, col, L), axis=1, keepdims=True)
        w_scr[...] = jnp.where(col == am, jnp.float32(jnp.inf), w)
        dms.append(jnp.sum(jnp.where(col == am, dist, 0.0), axis=1,
                           keepdims=True))
        nbrs.append(am)
        d2s.append(mv)
    nbr = jnp.concatenate(nbrs, axis=1)
    d2n_ref[...] = jnp.concatenate(d2s, axis=1)
    dmn_ref[...] = jnp.concatenate(dms, axis=1)
    nbr_ref[...] = nbr
    rowk = i0 + lax.broadcasted_iota(jnp.int32, (BLK, K), 0)
    flat_ref[...] = rowk * L + nbr
    ss_ref[...] = (rowk - nbr).astype(jnp.float32) / 100.0


def _knn(ca_pad, ca_t, dist2d):
    return pl.pallas_call(
        _knn_body,
        grid=(NBLK,),
        in_specs=[
            pl.BlockSpec((BLK, 8), lambda i: (i, 0)),
            pl.BlockSpec((8, L), lambda i: (0, 0)),
            pl.BlockSpec((BLK, L), lambda i: (i, 0)),
        ],
        out_specs=[pl.BlockSpec((BLK, K), lambda i: (i, 0))] * 5,
        out_shape=[
            jax.ShapeDtypeStruct((L, K), jnp.int32),
            jax.ShapeDtypeStruct((L, K), jnp.int32),
            jax.ShapeDtypeStruct((L, K), jnp.float32),
            jax.ShapeDtypeStruct((L, K), jnp.float32),
            jax.ShapeDtypeStruct((L, K), jnp.float32),
        ],
        scratch_shapes=[pltpu.VMEM((BLK, L), jnp.float32)],
    )(ca_pad, ca_t, dist2d)


def _edge_body(pairg_ref, d2n_ref, ss_ref, dm_ref, cen_ref, we_ref,
               be_ref, msa_ref, st_ref, wn_ref, bn_ref, e_ref, h_ref):
    pair_nbr = pairg_ref[...]
    d = jnp.sqrt(d2n_ref[...] + 1e-8)
    z = (d - cen_ref[...]) / jnp.float32(20.0 / D)
    rbf = jnp.exp(-z * z)
    e_in = jnp.concatenate([pair_nbr, rbf, ss_ref[...], dm_ref[...]], axis=1)
    e_ref[...] = jnp.dot(e_in, we_ref[...], precision=_PREC) + be_ref[...]
    hin = jnp.concatenate([msa_ref[...], st_ref[...]], axis=1)
    h0 = jnp.dot(hin, wn_ref[...], precision=_PREC) + bn_ref[...]
    h_ref[...] = jnp.concatenate([h0, jnp.zeros((BLK, D), jnp.float32)],
                                 axis=1)


def _edge_node(pairg, d2n_f, ss_f, dm_f, centers, we, be,
               msa0, state, wn, bn):
    g = BLK * K
    return pl.pallas_call(
        _edge_body,
        grid=(NBLK,),
        in_specs=[
            pl.BlockSpec((g, D), lambda i: (i, 0)),
            pl.BlockSpec((g, 1), lambda i: (i, 0)),
            pl.BlockSpec((g, 1), lambda i: (i, 0)),
            pl.BlockSpec((g, 1), lambda i: (i, 0)),
            pl.BlockSpec((1, D), lambda i: (0, 0)),
            pl.BlockSpec((2 * D + 2, D), lambda i: (0, 0)),
            pl.BlockSpec((1, D), lambda i: (0, 0)),
            pl.BlockSpec((BLK, D), lambda i: (i, 0)),
            pl.BlockSpec((BLK, D), lambda i: (i, 0)),
            pl.BlockSpec((2 * D, D), lambda i: (0, 0)),
            pl.BlockSpec((1, D), lambda i: (0, 0)),
        ],
        out_specs=[
            pl.BlockSpec((g, D), lambda i: (i, 0)),
            pl.BlockSpec((BLK, 128), lambda i: (i, 0)),
        ],
        out_shape=[
            jax.ShapeDtypeStruct((L * K, D), jnp.float32),
            jax.ShapeDtypeStruct((L, 128), jnp.float32),
        ],
    )(pairg, d2n_f, ss_f, dm_f, centers, we, be, msa0, state, wn, bn)


def _msg_body(h_ref, hn_ref, e_ref, wm_ref, bm_ref, ho_ref):
    g = BLK * K
    h = h_ref[:, :D]
    h_self = jnp.broadcast_to(h[:, None, :], (BLK, K, D)).reshape(g, D)
    m_in = jnp.concatenate([h_self, hn_ref[:, :D], e_ref[...]], axis=1)
    m = jnp.maximum(
        jnp.dot(m_in, wm_ref[...], precision=_PREC) + bm_ref[...], 0.0)
    mean = jnp.mean(m.reshape(BLK, K, D), axis=1)
    hnew = h + mean
    ho_ref[...] = jnp.concatenate([hnew, jnp.zeros((BLK, D), jnp.float32)],
                                  axis=1)


def _msg(h, hn, e, wm, bm):
    g = BLK * K
    return pl.pallas_call(
        _msg_body,
        grid=(NBLK,),
        in_specs=[
            pl.BlockSpec((BLK, 128), lambda i: (i, 0)),
            pl.BlockSpec((g, 128), lambda i: (i, 0)),
            pl.BlockSpec((g, D), lambda i: (i, 0)),
            pl.BlockSpec((3 * D, D), lambda i: (0, 0)),
            pl.BlockSpec((1, D), lambda i: (0, 0)),
        ],
        out_specs=pl.BlockSpec((BLK, 128), lambda i: (i, 0)),
        out_shape=jax.ShapeDtypeStruct((L, 128), jnp.float32),
    )(h, hn, e, wm, bm)


def _tail_body(h_ref, xyz_ref, st_ref, msa_ref, wx_ref, bx_ref, wst_ref,
               bst_ref, w1_ref, b1_ref, w2_ref, b2_ref, msel_ref,
               xo_ref, so_ref, ao_ref):
    h = h_ref[:, :D]
    disp = jnp.dot(h, wx_ref[...], precision=_PREC) + bx_ref[...]
    xo_ref[...] = xyz_ref[...] + 0.1 * disp
    sn = st_ref[...] + jnp.dot(h, wst_ref[...], precision=_PREC) + bst_ref[...]
    so_ref[...] = sn
    ain = jnp.concatenate([msa_ref[...], sn], axis=1)
    hid = jnp.maximum(jnp.dot(ain, w1_ref[...], precision=_PREC) + b1_ref[...], 0.0)
    al = jnp.dot(hid, w2_ref[...], precision=_PREC) + b2_ref[...]
    ps = jnp.dot(al * al, msel_ref[...], precision=_HI)
    ao_ref[...] = al / (jnp.sqrt(ps) + 1e-8)


def _tail(h, xyz9, state, msa0, wx, bx, wst, bst, w1, b1, w2, b2, msel):
    t2 = 2 * N_TORSION
    return pl.pallas_call(
        _tail_body,
        grid=(NBLK,),
        in_specs=[
            pl.BlockSpec((BLK, 128), lambda i: (i, 0)),
            pl.BlockSpec((BLK, 9), lambda i: (i, 0)),
            pl.BlockSpec((BLK, D), lambda i: (i, 0)),
            pl.BlockSpec((BLK, D), lambda i: (i, 0)),
            pl.BlockSpec((D, 9), lambda i: (0, 0)),
            pl.BlockSpec((1, 9), lambda i: (0, 0)),
            pl.BlockSpec((D, D), lambda i: (0, 0)),
            pl.BlockSpec((1, D), lambda i: (0, 0)),
            pl.BlockSpec((2 * D, D_HID), lambda i: (0, 0)),
            pl.BlockSpec((1, D_HID), lambda i: (0, 0)),
            pl.BlockSpec((D_HID, t2), lambda i: (0, 0)),
            pl.BlockSpec((1, t2), lambda i: (0, 0)),
            pl.BlockSpec((t2, t2), lambda i: (0, 0)),
        ],
        out_specs=[
            pl.BlockSpec((BLK, 9), lambda i: (i, 0)),
            pl.BlockSpec((BLK, D), lambda i: (i, 0)),
            pl.BlockSpec((BLK, t2), lambda i: (i, 0)),
        ],
        out_shape=[
            jax.ShapeDtypeStruct((L, 9), jnp.float32),
            jax.ShapeDtypeStruct((L, D), jnp.float32),
            jax.ShapeDtypeStruct((L, t2), jnp.float32),
        ],
    )(h, xyz9, state, msa0, wx, bx, wst, bst, w1, b1, w2, b2, msel)


def kernel(msa, pair, state, xyz, chirals, dist_matrix, is_atom, atom_frames,
           idx, bond_feats, W_node, b_node, W_edge, b_edge, W_msg, b_msg,
           W_xyz, b_xyz, W_state, b_state, W_sc1, b_sc1, W_sc2, b_sc2):
    msa0 = msa[0, 0]
    pair2d = pair.reshape(L * L, D)
    dist2d = dist_matrix.reshape(L, L)
    st = state[0]
    xyz9 = xyz.reshape(L, 9)

    centers = jnp.linspace(0.0, 20.0, D).reshape(1, D)
    be = b_edge.reshape(1, D)
    bn = b_node.reshape(1, D)
    bx = b_xyz.reshape(1, 9)
    bst = b_state.reshape(1, D)
    b1 = b_sc1.reshape(1, D_HID)
    b2 = b_sc2.reshape(1, 2 * N_TORSION)
    t2 = 2 * N_TORSION
    msel = (jnp.arange(t2)[:, None] // 2 == jnp.arange(t2)[None, :] // 2
            ).astype(jnp.float32)

    xyzs, alphas = [], []
    for _ in range(2):
        ca = xyz9[:, 3:6]
        ca_pad = jnp.pad(ca, ((0, 0), (0, 5)))
        ca_t = jnp.pad(ca.T, ((0, 5), (0, 0)))
        nbr, flat, d2n, ss, dmn = _knn(ca_pad, ca_t, dist2d)
        pairg = _gather_rows64(pair2d, flat.reshape(NW, NCH, CHUNK))
        e, h = _edge_node(pairg, d2n.reshape(L * K, 1),
                          ss.reshape(L * K, 1), dmn.reshape(L * K, 1),
                          centers, W_edge, be, msa0, st, W_node, bn)
        nidx3 = nbr.reshape(NW, NCH, CHUNK)
        for l in range(2):
            hn = _gather_rows128(h, nidx3)
            h = _msg(h, hn, e, W_msg[l], b_msg[l].reshape(1, D))
        xyz9, st, al = _tail(h, xyz9, st, msa0, W_xyz, bx, W_state, bst,
                             W_sc1, b1, W_sc2, b2, msel)
        xyzs.append(xyz9.reshape(1, L, 3, 3))
        alphas.append(al.reshape(1, L, N_TORSION, 2))

    return (jnp.stack(xyzs, axis=0), st[None], jnp.stack(alphas, axis=0))
